# baseline (speedup 1.0000x reference)
import functools

import jax
import jax.numpy as jnp
from jax import lax
from jax.experimental import pallas as pl
from jax.experimental.pallas import tpu as pltpu
from jax.experimental.pallas import tpu_sc as plsc

C = 16
RES = 512
NC = 2
NS = 16
NW = NC * NS
B = 256
L = 16
NG = 12 * B
PLANE_DIMS = ((1, 2), (0, 2), (0, 1))


def _floor_parts(coord):
    p = coord * RES - 0.5
    t = p.astype(jnp.int32)
    tf = t.astype(jnp.float32)
    neg = tf > p
    fl_i = jnp.where(neg, t - 1, t)
    fl_f = jnp.where(neg, tf - 1.0, tf)
    w = p - fl_f
    i0 = jnp.clip(fl_i, 0, RES - 1)
    i1 = jnp.minimum(i0 + 1, RES - 1)
    return i0, i1, w


def _splat(vec, lane):
    idx = jnp.full((L,), lane, jnp.int32)
    return jnp.take_along_axis(vec, idx, axis=0)


def _sc_body(n_pad, x_hbm, fm_hbm, out_hbm, xb, idxb, wb, rows, outb, sems):
    per_w = n_pad // NW
    n_chunks = per_w // B
    wid = lax.axis_index("s") * NC + lax.axis_index("c")
    lane_iota = lax.iota(jnp.int32, L)

    def stage_and_index(k, par):
        base = wid * per_w + k * B
        pltpu.sync_copy(x_hbm.at[pl.ds(base * 3, B * 3)], xb)
        io = par * (12 * B)
        wo = par * (6 * B)

        def idx_body(g, _):
            o = g * L
            coords = [
                plsc.load_gather(xb, [3 * lane_iota + (3 * o + d)])
                for d in range(3)
            ]
            for plane, (ud, vd) in enumerate(PLANE_DIMS):
                x0, x1, wx = _floor_parts(coords[ud])
                y0, y1, wy = _floor_parts(coords[vd])
                pbase = plane * RES * RES
                r0 = pbase + (y0 << 9)
                r1 = pbase + (y1 << 9)
                idxb[pl.ds(io + (4 * plane + 0) * B + o, L)] = r0 + x0
                idxb[pl.ds(io + (4 * plane + 1) * B + o, L)] = r0 + x1
                idxb[pl.ds(io + (4 * plane + 2) * B + o, L)] = r1 + x0
                idxb[pl.ds(io + (4 * plane + 3) * B + o, L)] = r1 + x1
                wb[pl.ds(wo + (2 * plane + 0) * B + o, L)] = wx
                wb[pl.ds(wo + (2 * plane + 1) * B + o, L)] = wy
            return ()

        lax.fori_loop(0, B // L, idx_body, ())

        for h in range(NG // 128):
            pltpu.async_copy(
                fm_hbm.at[idxb.at[pl.ds(io + h * 128, 128)]],
                rows.at[pl.ds(par * NG + h * 128, 128)],
                sems.at[par],
            )

    def wait_gathers(par):
        for h in range(NG // 128):
            pltpu.make_async_copy(
                fm_hbm.at[idxb.at[pl.ds(par * (12 * B) + h * 128, 128)]],
                rows.at[pl.ds(par * NG + h * 128, 128)],
                sems.at[par],
            ).wait()

    def blend_and_store(k, par):
        base = wid * per_w + k * B
        ro = par * NG
        wo = par * (6 * B)

        def blend_body(g, _):
            o = g * L
            wv = [wb[pl.ds(wo + i * B + o, L)] for i in range(6)]
            for p in range(L):
                pt = o + p
                for plane in range(3):
                    wx = _splat(wv[2 * plane + 0], p)
                    wy = _splat(wv[2 * plane + 1], p)
                    f00 = rows[ro + (4 * plane + 0) * B + pt, :]
                    f01 = rows[ro + (4 * plane + 1) * B + pt, :]
                    f10 = rows[ro + (4 * plane + 2) * B + pt, :]
                    f11 = rows[ro + (4 * plane + 3) * B + pt, :]
                    top = f00 + wx * (f01 - f00)
                    bot = f10 + wx * (f11 - f10)
                    outb[pl.ds(pt * 3 * C + plane * C, C)] = (
                        top + wy * (bot - top)
                    )
            return ()

        lax.fori_loop(0, B // L, blend_body, ())
        pltpu.sync_copy(outb, out_hbm.at[pl.ds(base * 3 * C, B * 3 * C)])

    assert n_chunks % 2 == 1
    stage_and_index(0, 0)

    def pair_body(j, _):
        k = 2 * j
        stage_and_index(k + 1, 1)
        wait_gathers(0)
        blend_and_store(k, 0)
        stage_and_index(k + 2, 0)
        wait_gathers(1)
        blend_and_store(k + 1, 1)
        return ()

    lax.fori_loop(0, (n_chunks - 1) // 2, pair_body, ())
    wait_gathers(0)
    blend_and_store(n_chunks - 1, 0)


@jax.jit
def kernel(x, fm):
    n = x.shape[0]
    per_w = -(-n // (NW * B)) * B
    n_pad = per_w * NW
    x_pad = jnp.zeros((n_pad, 3), jnp.float32).at[:n].set(x).reshape(-1)
    fm_flat = fm.reshape(3 * RES * RES, C)

    mesh = plsc.VectorSubcoreMesh(
        core_axis_name="c", subcore_axis_name="s", num_cores=NC, num_subcores=NS
    )
    out = pl.kernel(
        functools.partial(_sc_body, n_pad),
        out_type=jax.ShapeDtypeStruct((n_pad * 3 * C,), jnp.float32),
        mesh=mesh,
        scratch_types=[
            pltpu.VMEM((3 * B,), jnp.float32),
            pltpu.VMEM((2 * 12 * B,), jnp.int32),
            pltpu.VMEM((2 * 6 * B,), jnp.float32),
            pltpu.VMEM((2 * NG, C), jnp.float32),
            pltpu.VMEM((B * 3 * C,), jnp.float32),
            pltpu.SemaphoreType.DMA((2,)),
        ],
        compiler_params=pltpu.CompilerParams(
            use_tc_tiling_on_sc=False, needs_layout_passes=False
        ),
    )(x_pad, fm_flat)
    return out.reshape(n_pad, 3 * C)[:n]


# speedup vs baseline: 2.3480x; 2.3480x over previous
import functools

import jax
import jax.numpy as jnp
from jax import lax
from jax.experimental import pallas as pl
from jax.experimental.pallas import tpu as pltpu
from jax.experimental.pallas import tpu_sc as plsc

C = 16
RES = 512
NC = 2
NS = 16
NW = NC * NS
B = 256
L = 16
NG = 12 * B
PLANE_DIMS = ((1, 2), (0, 2), (0, 1))


def _floor_parts(coord):
    p = coord * RES - 0.5
    t = p.astype(jnp.int32)
    tf = t.astype(jnp.float32)
    neg = tf > p
    fl_i = jnp.where(neg, t - 1, t)
    fl_f = jnp.where(neg, tf - 1.0, tf)
    w = p - fl_f
    i0 = jnp.clip(fl_i, 0, RES - 1)
    i1 = jnp.minimum(i0 + 1, RES - 1)
    return i0, i1, w


def _splat(vec, lane):
    idx = jnp.full((L,), lane, jnp.int32)
    return jnp.take_along_axis(vec, idx, axis=0)


def _sc_body(n_pad, x_hbm, fm_hbm, out_hbm, xb, idxb, wb, rows, outb, sems):
    per_w = n_pad // NW
    n_chunks = per_w // B
    wid = lax.axis_index("s") * NC + lax.axis_index("c")

    def stage_and_index(k, par):
        base = wid * per_w + k * B
        for d in range(3):
            pltpu.sync_copy(
                x_hbm.at[pl.ds(d * n_pad + base, B)], xb.at[pl.ds(d * B, B)]
            )
        io = par * (12 * B)
        wo = par * (6 * B)

        def idx_body(g, _):
            o = g * L
            coords = [xb[pl.ds(d * B + o, L)] for d in range(3)]
            for plane, (ud, vd) in enumerate(PLANE_DIMS):
                x0, x1, wx = _floor_parts(coords[ud])
                y0, y1, wy = _floor_parts(coords[vd])
                pbase = plane * RES * RES
                r0 = pbase + (y0 << 9)
                r1 = pbase + (y1 << 9)
                idxb[pl.ds(io + (4 * plane + 0) * B + o, L)] = r0 + x0
                idxb[pl.ds(io + (4 * plane + 1) * B + o, L)] = r0 + x1
                idxb[pl.ds(io + (4 * plane + 2) * B + o, L)] = r1 + x0
                idxb[pl.ds(io + (4 * plane + 3) * B + o, L)] = r1 + x1
                wb[pl.ds(wo + (2 * plane + 0) * B + o, L)] = wx
                wb[pl.ds(wo + (2 * plane + 1) * B + o, L)] = wy
            return ()

        lax.fori_loop(0, B // L, idx_body, ())

        for h in range(NG // 128):
            pltpu.async_copy(
                fm_hbm.at[idxb.at[pl.ds(io + h * 128, 128)]],
                rows.at[pl.ds(par * NG + h * 128, 128)],
                sems.at[par],
            )

    def wait_gathers(par):
        for h in range(NG // 128):
            pltpu.make_async_copy(
                fm_hbm.at[idxb.at[pl.ds(par * (12 * B) + h * 128, 128)]],
                rows.at[pl.ds(par * NG + h * 128, 128)],
                sems.at[par],
            ).wait()

    def blend_and_store(k, par):
        base = wid * per_w + k * B
        ro = par * NG
        wo = par * (6 * B)

        def blend_body(g, _):
            o = g * L
            wv = [wb[pl.ds(wo + i * B + o, L)] for i in range(6)]
            for p in range(L):
                pt = o + p
                for plane in range(3):
                    wx = _splat(wv[2 * plane + 0], p)
                    wy = _splat(wv[2 * plane + 1], p)
                    f00 = rows[ro + (4 * plane + 0) * B + pt, :]
                    f01 = rows[ro + (4 * plane + 1) * B + pt, :]
                    f10 = rows[ro + (4 * plane + 2) * B + pt, :]
                    f11 = rows[ro + (4 * plane + 3) * B + pt, :]
                    top = f00 + wx * (f01 - f00)
                    bot = f10 + wx * (f11 - f10)
                    outb[pl.ds(pt * 3 * C + plane * C, C)] = (
                        top + wy * (bot - top)
                    )
            return ()

        lax.fori_loop(0, B // L, blend_body, ())
        pltpu.sync_copy(outb, out_hbm.at[pl.ds(base * 3 * C, B * 3 * C)])

    assert n_chunks % 2 == 1
    stage_and_index(0, 0)

    def pair_body(j, _):
        k = 2 * j
        stage_and_index(k + 1, 1)
        wait_gathers(0)
        blend_and_store(k, 0)
        stage_and_index(k + 2, 0)
        wait_gathers(1)
        blend_and_store(k + 1, 1)
        return ()

    lax.fori_loop(0, (n_chunks - 1) // 2, pair_body, ())
    wait_gathers(0)
    blend_and_store(n_chunks - 1, 0)


@jax.jit
def kernel(x, fm):
    n = x.shape[0]
    per_w = -(-n // (NW * B)) * B
    n_pad = per_w * NW
    x_pad = jnp.zeros((3, n_pad), jnp.float32).at[:, :n].set(x.T).reshape(-1)
    fm_flat = fm.reshape(3 * RES * RES, C)

    mesh = plsc.VectorSubcoreMesh(
        core_axis_name="c", subcore_axis_name="s", num_cores=NC, num_subcores=NS
    )
    out = pl.kernel(
        functools.partial(_sc_body, n_pad),
        out_type=jax.ShapeDtypeStruct((n_pad * 3 * C,), jnp.float32),
        mesh=mesh,
        scratch_types=[
            pltpu.VMEM((3 * B,), jnp.float32),
            pltpu.VMEM((2 * 12 * B,), jnp.int32),
            pltpu.VMEM((2 * 6 * B,), jnp.float32),
            pltpu.VMEM((2 * NG, C), jnp.float32),
            pltpu.VMEM((B * 3 * C,), jnp.float32),
            pltpu.SemaphoreType.DMA((2,)),
        ],
        compiler_params=pltpu.CompilerParams(use_tc_tiling_on_sc=False),
    )(x_pad, fm_flat)
    return out.reshape(n_pad, 3 * C)[:n]
